# baseline (speedup 1.0000x reference)
import jax
import jax.numpy as jnp
from jax.experimental import pallas as pl

B = 32; S = 256; IN = 128; D = 128; H = 8; L = 2; NC = 16; V = 16; FF = 512
DH = D // H


def _dot(a, b, dims=None):
    if dims is None:
        dims = (((1,), (0,)), ((), ()))
    return jax.lax.dot_general(a, b, dims, preferred_element_type=jnp.float32)


def _rowsum_strided(x):
    n = x.shape[-1]
    while n > 128:
        x = x[..., : n // 2] + x[..., n // 2:]
        n //= 2
    acc = x[..., 0:8]
    for t in range(1, n // 8):
        acc = acc + x[..., 8 * t:8 * t + 8]
    a = acc[..., :4] + acc[..., 4:]
    a = a[..., :2] + a[..., 2:]
    return a[..., :1] + a[..., 1:]


def _layernorm(x, g, b):
    m = _rowsum_strided(x) * jnp.float32(1.0 / 128.0)
    t = x - m
    v = _rowsum_strided(t * t) * jnp.float32(1.0 / 128.0)
    return t / jnp.sqrt(v + 1e-5) * g + b


def _att_head(sc, vh):
    m = sc.max(-1, keepdims=True)
    e = jnp.exp(sc - m)
    s = e.sum(-1, keepdims=True)
    return _dot(e, vh) / s


def _fwd_kernel(x_ref, Wenc_ref, benc_ref, Wq_ref, bq_ref, Wk_ref, bk_ref,
                Wv_ref, bv_ref, Wo_ref, bo_ref, W1_ref, b1_ref, W2_ref,
                b2_ref, ln1g_ref, ln1b_ref, ln2g_ref, ln2b_ref,
                ae_e1_ref, ae_e1b_ref, ae_e2_ref, ae_e2b_ref,
                ae_d1_ref, ae_d1b_ref, ae_d2_ref, ae_d2b_ref,
                itc_ref):
    xb = x_ref[0]
    h = _dot(xb, Wenc_ref[...]) + benc_ref[...][None, :]
    for l in range(L):
        q = _dot(h, Wq_ref[l]) + bq_ref[l][None, :]
        k = _dot(h, Wk_ref[l]) + bk_ref[l][None, :]
        v = _dot(h, Wv_ref[l]) + bv_ref[l][None, :]
        outs = []
        for hh in range(H):
            sl = slice(hh * DH, (hh + 1) * DH)
            sc = _dot(q[:, sl], k[:, sl], (((1,), (1,)), ((), ()))) / jnp.sqrt(
                jnp.float32(DH))
            outs.append(_att_head(sc, v[:, sl]))
        o = jnp.concatenate(outs, axis=-1)
        o = _dot(o, Wo_ref[l]) + bo_ref[l][None, :]
        h = _layernorm(h + o, ln1g_ref[l][None, :], ln1b_ref[l][None, :])
        f = _dot(jax.nn.relu(_dot(h, W1_ref[l]) + b1_ref[l][None, :]),
                 W2_ref[l]) + b2_ref[l][None, :]
        h = _layernorm(h + f, ln2g_ref[l][None, :], ln2b_ref[l][None, :])
    e = _dot(jax.nn.relu(_dot(h, ae_e1_ref[...]) + ae_e1b_ref[...][None, :]),
             ae_e2_ref[...]) + ae_e2b_ref[...][None, :]
    itc = _dot(jax.nn.relu(_dot(e, ae_d1_ref[...]) + ae_d1b_ref[...][None, :]),
               ae_d2_ref[...]) + ae_d2b_ref[...][None, :]
    itc_ref[0] = itc


SBLK = 8


def _assign_kernel(itct_ref, ys_ref, asg_ref, tot_ref):
    lab = ys_ref[...]
    part = jnp.zeros((1, 1), jnp.float32)
    asg_rows = []
    ic = jax.lax.broadcasted_iota(jnp.int32, (B, B, B), 1)
    icp = jax.lax.broadcasted_iota(jnp.int32, (B, B, B), 2)
    ic2 = jax.lax.broadcasted_iota(jnp.int32, (B, B), 1)
    ivc = jax.lax.broadcasted_iota(jnp.int32, (B, V), 1)
    for si in range(SBLK):
        a = itct_ref[si]
        diff = a[:, None, :] - a[None, :, :]
        dist = (diff * diff).sum(axis=-1)
        z = -dist
        zmax = z.max(axis=-1, keepdims=True)
        e = jnp.exp(z - zmax)
        p = e / _rowsum_strided(e)
        beats = ((p[:, None, :] > p[:, :, None])
                 | ((p[:, None, :] == p[:, :, None]) & (icp < ic)))
        rank = beats.astype(jnp.int32).sum(axis=2)
        maskc = ((rank == ic2) & (ic2 < NC)).astype(jnp.float32).sum(axis=-1)
        row_sum = dist.sum(axis=-1)
        part = part + (maskc * row_sum).sum(keepdims=True).reshape(1, 1)
        sel = (rank < NC).astype(jnp.float32)
        lab_oh = (lab[si][:, None] == ivc).astype(jnp.float32)
        counts = (sel[:, :, None] * lab_oh[None, :, :]).sum(axis=1)
        amax = counts.max(axis=-1, keepdims=True)
        asg_rows.append(jnp.where(counts == amax, ivc, V).min(axis=-1))
    asg_ref[...] = jnp.stack(asg_rows).astype(jnp.int32)

    @pl.when(pl.program_id(0) == 0)
    def _():
        tot_ref[...] = jnp.zeros((1, 1), jnp.float32)

    tot_ref[...] += part


def _cont_kernel(asg_ref, yt_ref, cont_ref):
    a = asg_ref[...]
    t = yt_ref[...]
    iv = jax.lax.broadcasted_iota(jnp.int32, (S, B, V), 2)
    t_oh = (t[:, :, None] == iv).astype(jnp.float32)
    rows = []
    for pcls in range(V):
        mp = (a == pcls).astype(jnp.float32)[:, :, None]
        rows.append((mp * t_oh).sum(axis=0).sum(axis=0))
    cont_ref[...] = jnp.stack(rows)


def _full(shape):
    nd = len(shape)
    return pl.BlockSpec(shape, lambda b, _n=nd: (0,) * _n)


def kernel(x, y, Wenc, benc, Wq, bq, Wk, bk, Wv, bv, Wo, bo, W1, b1, W2, b2,
           ln1g, ln1b, ln2g, ln2b, ae_e1, ae_e1b, ae_e2, ae_e2b,
           ae_d1, ae_d1b, ae_d2, ae_d2b):
    itc = pl.pallas_call(
        _fwd_kernel,
        grid=(B,),
        in_specs=[
            pl.BlockSpec((1, S, IN), lambda b: (b, 0, 0)),
            _full(Wenc.shape), _full(benc.shape),
            _full(Wq.shape), _full(bq.shape),
            _full(Wk.shape), _full(bk.shape),
            _full(Wv.shape), _full(bv.shape),
            _full(Wo.shape), _full(bo.shape),
            _full(W1.shape), _full(b1.shape),
            _full(W2.shape), _full(b2.shape),
            _full(ln1g.shape), _full(ln1b.shape),
            _full(ln2g.shape), _full(ln2b.shape),
            _full(ae_e1.shape), _full(ae_e1b.shape),
            _full(ae_e2.shape), _full(ae_e2b.shape),
            _full(ae_d1.shape), _full(ae_d1b.shape),
            _full(ae_d2.shape), _full(ae_d2b.shape),
        ],
        out_specs=pl.BlockSpec((1, S, D), lambda b: (b, 0, 0)),
        out_shape=jax.ShapeDtypeStruct((B, S, D), jnp.float32),
    )(x, Wenc, benc, Wq, bq, Wk, bk, Wv, bv, Wo, bo, W1, b1, W2, b2,
      ln1g, ln1b, ln2g, ln2b, ae_e1, ae_e1b, ae_e2, ae_e2b,
      ae_d1, ae_d1b, ae_d2, ae_d2b)

    itct = itc.transpose(1, 0, 2)
    y2 = y[:, :, 0]
    ys = y2.T
    assigned, tot = pl.pallas_call(
        _assign_kernel,
        grid=(S // SBLK,),
        in_specs=[
            pl.BlockSpec((SBLK, B, D), lambda i: (i, 0, 0)),
            pl.BlockSpec((SBLK, B), lambda i: (i, 0)),
        ],
        out_specs=[
            pl.BlockSpec((SBLK, B), lambda i: (i, 0)),
            pl.BlockSpec((1, 1), lambda i: (0, 0)),
        ],
        out_shape=[
            jax.ShapeDtypeStruct((S, B), jnp.int32),
            jax.ShapeDtypeStruct((1, 1), jnp.float32),
        ],
    )(itct, ys)

    ytrue2 = y.reshape(-1).reshape(S, B)
    cont = pl.pallas_call(
        _cont_kernel,
        out_shape=jax.ShapeDtypeStruct((V, V), jnp.float32),
    )(assigned, ytrue2)

    c2 = lambda m: m * (m - 1.0) / 2.0
    sumc = c2(cont).sum()
    arow = c2(cont.sum(1)).sum()
    bcol = c2(cont.sum(0)).sum()
    totpairs = c2(jnp.asarray(S * B, jnp.float32))
    exp = arow * bcol / totpairs
    maxi = (arow + bcol) / 2.0
    ari = (sumc - exp) / (maxi - exp + 1e-12)

    return (tot[0, 0], ari, assigned, itc)


# speedup vs baseline: 1.0319x; 1.0319x over previous
import jax
import jax.numpy as jnp
from jax.experimental import pallas as pl

B = 32; S = 256; IN = 128; D = 128; H = 8; L = 2; NC = 16; V = 16; FF = 512
DH = D // H


def _dot(a, b, dims=None):
    if dims is None:
        dims = (((1,), (0,)), ((), ()))
    return jax.lax.dot_general(a, b, dims, preferred_element_type=jnp.float32)


def _rowsum_strided(x):
    n = x.shape[-1]
    while n > 128:
        x = x[..., : n // 2] + x[..., n // 2:]
        n //= 2
    acc = x[..., 0:8]
    for t in range(1, n // 8):
        acc = acc + x[..., 8 * t:8 * t + 8]
    a = acc[..., :4] + acc[..., 4:]
    a = a[..., :2] + a[..., 2:]
    return a[..., :1] + a[..., 1:]


def _layernorm(x, g, b):
    m = _rowsum_strided(x) * jnp.float32(1.0 / 128.0)
    t = x - m
    v = _rowsum_strided(t * t) * jnp.float32(1.0 / 128.0)
    return t / jnp.sqrt(v + 1e-5) * g + b


def _att_head(sc, vh):
    m = sc.max(-1, keepdims=True)
    e = jnp.exp(sc - m)
    s = e.sum(-1, keepdims=True)
    return _dot(e, vh) / s


def _fwd_kernel(x_ref, Wenc_ref, benc_ref, Wq_ref, bq_ref, Wk_ref, bk_ref,
                Wv_ref, bv_ref, Wo_ref, bo_ref, W1_ref, b1_ref, W2_ref,
                b2_ref, ln1g_ref, ln1b_ref, ln2g_ref, ln2b_ref,
                ae_e1_ref, ae_e1b_ref, ae_e2_ref, ae_e2b_ref,
                ae_d1_ref, ae_d1b_ref, ae_d2_ref, ae_d2b_ref,
                itc_ref):
    for bi in range(NB):
        xb = x_ref[bi]
        h = _dot(xb, Wenc_ref[...]) + benc_ref[...][None, :]
        for l in range(L):
            q = _dot(h, Wq_ref[l]) + bq_ref[l][None, :]
            k = _dot(h, Wk_ref[l]) + bk_ref[l][None, :]
            v = _dot(h, Wv_ref[l]) + bv_ref[l][None, :]
            outs = []
            for hh in range(H):
                sl = slice(hh * DH, (hh + 1) * DH)
                sc = _dot(q[:, sl], k[:, sl],
                          (((1,), (1,)), ((), ()))) / jnp.sqrt(jnp.float32(DH))
                outs.append(_att_head(sc, v[:, sl]))
            o = jnp.concatenate(outs, axis=-1)
            o = _dot(o, Wo_ref[l]) + bo_ref[l][None, :]
            h = _layernorm(h + o, ln1g_ref[l][None, :], ln1b_ref[l][None, :])
            f = _dot(jax.nn.relu(_dot(h, W1_ref[l]) + b1_ref[l][None, :]),
                     W2_ref[l]) + b2_ref[l][None, :]
            h = _layernorm(h + f, ln2g_ref[l][None, :], ln2b_ref[l][None, :])
        e = _dot(jax.nn.relu(_dot(h, ae_e1_ref[...]) + ae_e1b_ref[...][None, :]),
                 ae_e2_ref[...]) + ae_e2b_ref[...][None, :]
        itc = _dot(jax.nn.relu(_dot(e, ae_d1_ref[...]) + ae_d1b_ref[...][None, :]),
                   ae_d2_ref[...]) + ae_d2b_ref[...][None, :]
        itc_ref[bi] = itc


NB = 4
SBLK = 32


def _assign_kernel(itct_ref, ys_ref, asg_ref, tot_ref):
    lab = ys_ref[...]
    part = jnp.zeros((1, 1), jnp.float32)
    asg_rows = []
    ic = jax.lax.broadcasted_iota(jnp.int32, (B, B, B), 1)
    icp = jax.lax.broadcasted_iota(jnp.int32, (B, B, B), 2)
    ic2 = jax.lax.broadcasted_iota(jnp.int32, (B, B), 1)
    ivc = jax.lax.broadcasted_iota(jnp.int32, (B, V), 1)
    for si in range(SBLK):
        a = itct_ref[si]
        diff = a[:, None, :] - a[None, :, :]
        dist = (diff * diff).sum(axis=-1)
        z = -dist
        zmax = z.max(axis=-1, keepdims=True)
        e = jnp.exp(z - zmax)
        p = e / _rowsum_strided(e)
        beats = ((p[:, None, :] > p[:, :, None])
                 | ((p[:, None, :] == p[:, :, None]) & (icp < ic)))
        rank = beats.astype(jnp.int32).sum(axis=2)
        maskc = ((rank == ic2) & (ic2 < NC)).astype(jnp.float32).sum(axis=-1)
        row_sum = dist.sum(axis=-1)
        part = part + (maskc * row_sum).sum(keepdims=True).reshape(1, 1)
        sel = (rank < NC).astype(jnp.float32)
        lab_oh = (lab[si][:, None] == ivc).astype(jnp.float32)
        counts = (sel[:, :, None] * lab_oh[None, :, :]).sum(axis=1)
        amax = counts.max(axis=-1, keepdims=True)
        asg_rows.append(jnp.where(counts == amax, ivc, V).min(axis=-1))
    asg_ref[...] = jnp.stack(asg_rows).astype(jnp.int32)

    @pl.when(pl.program_id(0) == 0)
    def _():
        tot_ref[...] = jnp.zeros((1, 1), jnp.float32)

    tot_ref[...] += part


def _cont_kernel(asg_ref, yt_ref, cont_ref):
    a = asg_ref[...]
    t = yt_ref[...]
    iv = jax.lax.broadcasted_iota(jnp.int32, (S, B, V), 2)
    t_oh = (t[:, :, None] == iv).astype(jnp.float32)
    rows = []
    for pcls in range(V):
        mp = (a == pcls).astype(jnp.float32)[:, :, None]
        rows.append((mp * t_oh).sum(axis=0).sum(axis=0))
    cont_ref[...] = jnp.stack(rows)


def _full(shape):
    nd = len(shape)
    return pl.BlockSpec(shape, lambda b, _n=nd: (0,) * _n)


def kernel(x, y, Wenc, benc, Wq, bq, Wk, bk, Wv, bv, Wo, bo, W1, b1, W2, b2,
           ln1g, ln1b, ln2g, ln2b, ae_e1, ae_e1b, ae_e2, ae_e2b,
           ae_d1, ae_d1b, ae_d2, ae_d2b):
    itc = pl.pallas_call(
        _fwd_kernel,
        grid=(B // NB,),
        in_specs=[
            pl.BlockSpec((NB, S, IN), lambda b: (b, 0, 0)),
            _full(Wenc.shape), _full(benc.shape),
            _full(Wq.shape), _full(bq.shape),
            _full(Wk.shape), _full(bk.shape),
            _full(Wv.shape), _full(bv.shape),
            _full(Wo.shape), _full(bo.shape),
            _full(W1.shape), _full(b1.shape),
            _full(W2.shape), _full(b2.shape),
            _full(ln1g.shape), _full(ln1b.shape),
            _full(ln2g.shape), _full(ln2b.shape),
            _full(ae_e1.shape), _full(ae_e1b.shape),
            _full(ae_e2.shape), _full(ae_e2b.shape),
            _full(ae_d1.shape), _full(ae_d1b.shape),
            _full(ae_d2.shape), _full(ae_d2b.shape),
        ],
        out_specs=pl.BlockSpec((NB, S, D), lambda b: (b, 0, 0)),
        out_shape=jax.ShapeDtypeStruct((B, S, D), jnp.float32),
    )(x, Wenc, benc, Wq, bq, Wk, bk, Wv, bv, Wo, bo, W1, b1, W2, b2,
      ln1g, ln1b, ln2g, ln2b, ae_e1, ae_e1b, ae_e2, ae_e2b,
      ae_d1, ae_d1b, ae_d2, ae_d2b)

    itct = itc.transpose(1, 0, 2)
    y2 = y[:, :, 0]
    ys = y2.T
    assigned, tot = pl.pallas_call(
        _assign_kernel,
        grid=(S // SBLK,),
        in_specs=[
            pl.BlockSpec((SBLK, B, D), lambda i: (i, 0, 0)),
            pl.BlockSpec((SBLK, B), lambda i: (i, 0)),
        ],
        out_specs=[
            pl.BlockSpec((SBLK, B), lambda i: (i, 0)),
            pl.BlockSpec((1, 1), lambda i: (0, 0)),
        ],
        out_shape=[
            jax.ShapeDtypeStruct((S, B), jnp.int32),
            jax.ShapeDtypeStruct((1, 1), jnp.float32),
        ],
    )(itct, ys)

    ytrue2 = y.reshape(-1).reshape(S, B)
    cont = pl.pallas_call(
        _cont_kernel,
        out_shape=jax.ShapeDtypeStruct((V, V), jnp.float32),
    )(assigned, ytrue2)

    c2 = lambda m: m * (m - 1.0) / 2.0
    sumc = c2(cont).sum()
    arow = c2(cont.sum(1)).sum()
    bcol = c2(cont.sum(0)).sum()
    totpairs = c2(jnp.asarray(S * B, jnp.float32))
    exp = arow * bcol / totpairs
    maxi = (arow + bcol) / 2.0
    ari = (sumc - exp) / (maxi - exp + 1e-12)

    return (tot[0, 0], ari, assigned, itc)
